# Initial kernel scaffold; baseline (speedup 1.0000x reference)
import jax
import jax.numpy as jnp
from jax.experimental import pallas as pl


def kernel(x, edge_index, batch, W1, b1, W2, b2, Wl, bl):
    raise NotImplementedError("write your pallas kernel here")


# speedup vs baseline: 39.8084x; 39.8084x over previous
import functools

import jax
import jax.numpy as jnp
from jax import lax
from jax.experimental import pallas as pl
from jax.experimental.pallas import tpu as pltpu
from jax.experimental.pallas import tpu_sc as plsc

N_NODES = 100000
N_EDGES = 6400000
NUM_GRAPHS = 128
F1, F2, FO = 8, 8, 2

NC, NS = 2, 16
NW = NC * NS
NP = 100096
RPT = NP // NS

OPS_PER_CHUNK = 8
CHUNK = OPS_PER_CHUNK * 128
NCHUNK = N_EDGES // CHUNK
TRIPS = (NCHUNK + NW - 1) // NW

BLK = RPT
TC_GRID = NP // BLK


def _sc_count_body(col_hbm, ones_hbm, zeros_hbm, out_hbm, cnt_sp, idx_v,
                   ones_v, cp_v):
    cid = lax.axis_index("c")
    sid = lax.axis_index("s")
    wid = sid * NC + cid
    base = sid * RPT
    pltpu.sync_copy(zeros_hbm.at[pl.ds(base, RPT)], cp_v)
    pltpu.sync_copy(cp_v, cnt_sp.at[pl.ds(base, RPT)])
    pltpu.sync_copy(ones_hbm, ones_v)
    plsc.subcore_barrier()

    def trip(k, carry):
        chunk = wid + k * NW

        @pl.when(chunk < NCHUNK)
        def _():
            pltpu.sync_copy(col_hbm.at[chunk], idx_v)
            for j in range(OPS_PER_CHUNK):
                pltpu.sync_copy(ones_v, cnt_sp.at[idx_v.at[j]], add=True)

        return carry

    lax.fori_loop(0, TRIPS, trip, 0)
    plsc.subcore_barrier()
    pltpu.sync_copy(cnt_sp.at[pl.ds(base, RPT)], cp_v)
    pltpu.sync_copy(cp_v, out_hbm.at[pl.ds(cid * NP + base, RPT)])


_sc_count = pl.kernel(
    _sc_count_body,
    out_type=jax.ShapeDtypeStruct((NC * NP,), jnp.float32),
    mesh=plsc.VectorSubcoreMesh(core_axis_name="c", subcore_axis_name="s"),
    compiler_params=pltpu.CompilerParams(use_tc_tiling_on_sc=False),
    scratch_types=[
        pltpu.VMEM_SHARED((NP,), jnp.float32),
        pltpu.VMEM((OPS_PER_CHUNK, 128), jnp.int32),
        pltpu.VMEM((128,), jnp.float32),
        pltpu.VMEM((RPT,), jnp.float32),
    ],
)


def _sc_edge_body(row_hbm, col_hbm, g_hbm, zeros_hbm, out_hbm, agg_sp, row_v,
                  col_v, msg_v, cp_v):
    cid = lax.axis_index("c")
    sid = lax.axis_index("s")
    wid = sid * NC + cid
    base = sid * RPT
    pltpu.sync_copy(zeros_hbm.at[pl.ds(base, RPT)], cp_v)
    pltpu.sync_copy(cp_v, agg_sp.at[pl.ds(base, RPT)])
    plsc.subcore_barrier()

    def trip(k, carry):
        chunk = wid + k * NW

        @pl.when(chunk < NCHUNK)
        def _():
            pltpu.sync_copy(row_hbm.at[chunk], row_v)
            pltpu.sync_copy(col_hbm.at[chunk], col_v)
            for j in range(OPS_PER_CHUNK):
                pltpu.sync_copy(g_hbm.at[row_v.at[j]], msg_v.at[j])
                pltpu.sync_copy(msg_v.at[j], agg_sp.at[col_v.at[j]], add=True)

        return carry

    lax.fori_loop(0, TRIPS, trip, 0)
    plsc.subcore_barrier()
    pltpu.sync_copy(agg_sp.at[pl.ds(base, RPT)], cp_v)
    pltpu.sync_copy(cp_v, out_hbm.at[pl.ds(cid * NP + base, RPT)])


_sc_edge = pl.kernel(
    _sc_edge_body,
    out_type=jax.ShapeDtypeStruct((NC * NP, F1), jnp.float32),
    mesh=plsc.VectorSubcoreMesh(core_axis_name="c", subcore_axis_name="s"),
    compiler_params=pltpu.CompilerParams(use_tc_tiling_on_sc=False),
    scratch_types=[
        pltpu.VMEM_SHARED((NP, F1), jnp.float32),
        pltpu.VMEM((OPS_PER_CHUNK, 128), jnp.int32),
        pltpu.VMEM((OPS_PER_CHUNK, 128), jnp.int32),
        pltpu.VMEM((OPS_PER_CHUNK, 128, F1), jnp.float32),
        pltpu.VMEM((RPT, F1), jnp.float32),
    ],
)


def _tc1_body(cnta, cntb, x, w1, dinv_o, h1_o, g1_o):
    cnt = cnta[...] + cntb[...] + 1.0
    dinv = lax.rsqrt(cnt)
    h1 = jnp.dot(x[...], w1[...], preferred_element_type=jnp.float32)
    dinv_o[...] = dinv
    h1_o[...] = h1
    g1_o[...] = h1 * dinv


def _tc2_body(agga, aggb, h1, dinv, w2, b1, h2_o, g2_o):
    dv = dinv[...]
    pre = (agga[...] + aggb[...]) * dv + h1[...] * (dv * dv) + b1[...]
    act = jnp.maximum(pre, 0.0)
    h2 = jnp.dot(act, w2[...], preferred_element_type=jnp.float32)
    h2_o[...] = h2
    g2_o[...] = h2 * dv


def _tc3_body(agga, aggb, h2, dinv, b2, batch, wl, bl, out_o, sums, cnts):
    i = pl.program_id(0)
    dv = dinv[...]
    z = jnp.maximum((agga[...] + aggb[...]) * dv + h2[...] * (dv * dv)
                    + b2[...], 0.0)
    onehot = (batch[...] == lax.broadcasted_iota(
        jnp.int32, (BLK, NUM_GRAPHS), 1)).astype(jnp.float32)
    dims = (((0,), (0,)), ((), ()))
    psum = lax.dot_general(onehot, z, dims,
                           preferred_element_type=jnp.float32)
    pcnt = lax.dot_general(onehot, jnp.ones((BLK, 1), jnp.float32), dims,
                           preferred_element_type=jnp.float32)

    @pl.when(i == 0)
    def _():
        sums[...] = psum
        cnts[...] = pcnt

    @pl.when(i > 0)
    def _():
        sums[...] += psum
        cnts[...] += pcnt

    @pl.when(i == TC_GRID - 1)
    def _():
        pooled = sums[...] / jnp.maximum(cnts[...], 1.0)
        out_o[...] = jnp.dot(pooled, wl[...],
                             preferred_element_type=jnp.float32) + bl[...]


def _row_spec(cols):
    return pl.BlockSpec((BLK, cols), lambda i: (i, 0))


def _full_spec(r, c):
    return pl.BlockSpec((r, c), lambda i: (0, 0))


_tc1 = pl.pallas_call(
    _tc1_body,
    grid=(TC_GRID,),
    in_specs=[_row_spec(1), _row_spec(1), _row_spec(3), _full_spec(3, F1)],
    out_specs=[_row_spec(1), _row_spec(F1), _row_spec(F1)],
    out_shape=[
        jax.ShapeDtypeStruct((NP, 1), jnp.float32),
        jax.ShapeDtypeStruct((NP, F1), jnp.float32),
        jax.ShapeDtypeStruct((NP, F1), jnp.float32),
    ],
)

_tc2 = pl.pallas_call(
    _tc2_body,
    grid=(TC_GRID,),
    in_specs=[_row_spec(F1), _row_spec(F1), _row_spec(F1), _row_spec(1),
              _full_spec(F1, F2), _full_spec(1, F2)],
    out_specs=[_row_spec(F2), _row_spec(F2)],
    out_shape=[
        jax.ShapeDtypeStruct((NP, F2), jnp.float32),
        jax.ShapeDtypeStruct((NP, F2), jnp.float32),
    ],
)

_tc3 = pl.pallas_call(
    _tc3_body,
    grid=(TC_GRID,),
    in_specs=[_row_spec(F2), _row_spec(F2), _row_spec(F2), _row_spec(1),
              _full_spec(1, F2), _row_spec(1), _full_spec(F2, FO),
              _full_spec(1, FO)],
    out_specs=_full_spec(NUM_GRAPHS, FO),
    out_shape=jax.ShapeDtypeStruct((NUM_GRAPHS, FO), jnp.float32),
    scratch_shapes=[
        pltpu.VMEM((NUM_GRAPHS, F2), jnp.float32),
        pltpu.VMEM((NUM_GRAPHS, 1), jnp.float32),
    ],
)


def kernel(x, edge_index, batch, W1, b1, W2, b2, Wl, bl):
    pad = NP - N_NODES
    xp = jnp.pad(x, ((0, pad), (0, 0)))
    batchp = jnp.pad(batch, (0, pad), constant_values=NUM_GRAPHS)
    batchp = batchp.reshape(NP, 1)
    row3 = edge_index[0].reshape(NCHUNK, OPS_PER_CHUNK, 128)
    col3 = edge_index[1].reshape(NCHUNK, OPS_PER_CHUNK, 128)
    ones128 = jnp.ones((128,), jnp.float32)
    zeros1 = jnp.zeros((NP,), jnp.float32)
    zeros2 = jnp.zeros((NP, F1), jnp.float32)
    b1r = b1.reshape(1, F1)
    b2r = b2.reshape(1, F2)
    blr = bl.reshape(1, FO)

    cnt2 = _sc_count(col3, ones128, zeros1).reshape(NC, NP, 1)
    dinv, h1, g1 = _tc1(cnt2[0], cnt2[1], xp, W1)
    agg1 = _sc_edge(row3, col3, g1, zeros2).reshape(NC, NP, F1)
    h2, g2 = _tc2(agg1[0], agg1[1], h1, dinv, W2, b1r)
    agg2 = _sc_edge(row3, col3, g2, zeros2).reshape(NC, NP, F1)
    return _tc3(agg2[0], agg2[1], h2, dinv, b2r, batchp, Wl, blr)


# speedup vs baseline: 83.5661x; 2.0992x over previous
import jax
import jax.numpy as jnp
from jax import lax
from jax.experimental import pallas as pl
from jax.experimental.pallas import tpu as pltpu
from jax.experimental.pallas import tpu_sc as plsc

N_NODES = 100000
N_EDGES = 6400000
NUM_GRAPHS = 128
F1, F2, FO = 8, 8, 2

NC, NS = 2, 16
NW = NC * NS
NP = 100096
RPT = NP // NS

OPS_PER_CHUNK = 8
CHUNK = OPS_PER_CHUNK * 128
TRIPS = 196
NCHUNK = NW * TRIPS
E_PAD = NCHUNK * CHUNK
BLK = RPT
TC_GRID = NP // BLK


def _sc_count_body(col_hbm, ones_hbm, zeros_hbm, out_hbm, cnt_sp, idx_v,
                   ones_v, cp_v, isem, ssem):
    cid = lax.axis_index("c")
    sid = lax.axis_index("s")
    wid = sid * NC + cid
    base = sid * RPT
    pltpu.sync_copy(zeros_hbm.at[pl.ds(base, RPT)], cp_v)
    pltpu.sync_copy(cp_v, cnt_sp.at[pl.ds(base, RPT)])
    pltpu.sync_copy(ones_hbm, ones_v)
    plsc.subcore_barrier()

    pltpu.async_copy(col_hbm.at[wid], idx_v.at[0], isem)

    def trip(k, carry):
        b = k & 1
        ck = wid + k * NW
        pltpu.make_async_copy(col_hbm.at[ck], idx_v.at[b], isem).wait()

        @pl.when(k > 0)
        def _():
            for j in range(OPS_PER_CHUNK):
                pltpu.make_async_copy(
                    ones_v, cnt_sp.at[idx_v.at[1 - b, j]], ssem).wait()

        @pl.when(k < TRIPS - 1)
        def _():
            pltpu.async_copy(col_hbm.at[ck + NW], idx_v.at[1 - b], isem)

        for j in range(OPS_PER_CHUNK):
            pltpu.async_copy(ones_v, cnt_sp.at[idx_v.at[b, j]], ssem,
                             add=True)
        return carry

    lax.fori_loop(0, TRIPS, trip, 0)
    bl_ = (TRIPS - 1) & 1
    for j in range(OPS_PER_CHUNK):
        pltpu.make_async_copy(ones_v, cnt_sp.at[idx_v.at[bl_, j]], ssem).wait()
    plsc.subcore_barrier()
    pltpu.sync_copy(cnt_sp.at[pl.ds(base, RPT)], cp_v)
    pltpu.sync_copy(cp_v, out_hbm.at[pl.ds(cid * NP + base, RPT)])


_sc_count = pl.kernel(
    _sc_count_body,
    out_type=jax.ShapeDtypeStruct((NC * NP,), jnp.float32),
    mesh=plsc.VectorSubcoreMesh(core_axis_name="c", subcore_axis_name="s"),
    compiler_params=pltpu.CompilerParams(use_tc_tiling_on_sc=False),
    scratch_types=[
        pltpu.VMEM_SHARED((NP,), jnp.float32),
        pltpu.VMEM((2, OPS_PER_CHUNK, 128), jnp.int32),
        pltpu.VMEM((128,), jnp.float32),
        pltpu.VMEM((RPT,), jnp.float32),
        pltpu.SemaphoreType.DMA,
        pltpu.SemaphoreType.DMA,
    ],
)


def _sc_edge_body(row_hbm, col_hbm, g_hbm, zeros_hbm, out_hbm, agg_sp, row_v,
                  col_v, msg_v, cp_v, isem, gsem, ssem):
    cid = lax.axis_index("c")
    sid = lax.axis_index("s")
    wid = sid * NC + cid
    base = sid * RPT
    pltpu.sync_copy(zeros_hbm.at[pl.ds(base, RPT)], cp_v)
    pltpu.sync_copy(cp_v, agg_sp.at[pl.ds(base, RPT)])
    plsc.subcore_barrier()

    pltpu.async_copy(row_hbm.at[wid], row_v.at[0], isem)
    pltpu.async_copy(col_hbm.at[wid], col_v.at[0], isem)

    def trip(k, carry):
        b = k & 1
        ck = wid + k * NW
        pltpu.make_async_copy(row_hbm.at[ck], row_v.at[b], isem).wait()
        pltpu.make_async_copy(col_hbm.at[ck], col_v.at[b], isem).wait()

        gds = [
            pltpu.async_copy(g_hbm.at[row_v.at[b, j]], msg_v.at[b, j], gsem)
            for j in range(OPS_PER_CHUNK)
        ]

        @pl.when(k > 0)
        def _():
            for j in range(OPS_PER_CHUNK):
                pltpu.make_async_copy(
                    msg_v.at[1 - b, j],
                    agg_sp.at[col_v.at[1 - b, j]], ssem).wait()

        @pl.when(k < TRIPS - 1)
        def _():
            pltpu.async_copy(row_hbm.at[ck + NW], row_v.at[1 - b], isem)
            pltpu.async_copy(col_hbm.at[ck + NW], col_v.at[1 - b], isem)

        for j in range(OPS_PER_CHUNK):
            gds[j].wait()
        for j in range(OPS_PER_CHUNK):
            pltpu.async_copy(msg_v.at[b, j], agg_sp.at[col_v.at[b, j]], ssem,
                             add=True)
        return carry

    lax.fori_loop(0, TRIPS, trip, 0)
    bl_ = (TRIPS - 1) & 1
    for j in range(OPS_PER_CHUNK):
        pltpu.make_async_copy(msg_v.at[bl_, j],
                              agg_sp.at[col_v.at[bl_, j]], ssem).wait()
    plsc.subcore_barrier()
    pltpu.sync_copy(agg_sp.at[pl.ds(base, RPT)], cp_v)
    pltpu.sync_copy(cp_v, out_hbm.at[pl.ds(cid * NP + base, RPT)])


_sc_edge = pl.kernel(
    _sc_edge_body,
    out_type=jax.ShapeDtypeStruct((NC * NP, F1), jnp.float32),
    mesh=plsc.VectorSubcoreMesh(core_axis_name="c", subcore_axis_name="s"),
    compiler_params=pltpu.CompilerParams(use_tc_tiling_on_sc=False),
    scratch_types=[
        pltpu.VMEM_SHARED((NP, F1), jnp.float32),
        pltpu.VMEM((2, OPS_PER_CHUNK, 128), jnp.int32),
        pltpu.VMEM((2, OPS_PER_CHUNK, 128), jnp.int32),
        pltpu.VMEM((2, OPS_PER_CHUNK, 128, F1), jnp.float32),
        pltpu.VMEM((RPT, F1), jnp.float32),
        pltpu.SemaphoreType.DMA,
        pltpu.SemaphoreType.DMA,
        pltpu.SemaphoreType.DMA,
    ],
)


def _tc1_body(cnta, cntb, x, w1, dinv_o, h1_o, g1_o):
    cnt = cnta[...] + cntb[...] + 1.0
    dinv = lax.rsqrt(cnt)
    h1 = jnp.dot(x[...], w1[...], preferred_element_type=jnp.float32)
    dinv_o[...] = dinv
    h1_o[...] = h1
    g1_o[...] = h1 * dinv


def _tc2_body(agga, aggb, h1, dinv, w2, b1, h2_o, g2_o):
    dv = dinv[...]
    pre = (agga[...] + aggb[...]) * dv + h1[...] * (dv * dv) + b1[...]
    act = jnp.maximum(pre, 0.0)
    h2 = jnp.dot(act, w2[...], preferred_element_type=jnp.float32)
    h2_o[...] = h2
    g2_o[...] = h2 * dv


def _tc3_body(agga, aggb, h2, dinv, b2, batch, wl, bl, out_o, sums, cnts):
    i = pl.program_id(0)
    dv = dinv[...]
    z = jnp.maximum((agga[...] + aggb[...]) * dv + h2[...] * (dv * dv)
                    + b2[...], 0.0)
    onehot = (batch[...] == lax.broadcasted_iota(
        jnp.int32, (BLK, NUM_GRAPHS), 1)).astype(jnp.float32)
    dims = (((0,), (0,)), ((), ()))
    psum = lax.dot_general(onehot, z, dims,
                           preferred_element_type=jnp.float32)
    pcnt = lax.dot_general(onehot, jnp.ones((BLK, 1), jnp.float32), dims,
                           preferred_element_type=jnp.float32)

    @pl.when(i == 0)
    def _():
        sums[...] = psum
        cnts[...] = pcnt

    @pl.when(i > 0)
    def _():
        sums[...] += psum
        cnts[...] += pcnt

    @pl.when(i == TC_GRID - 1)
    def _():
        pooled = sums[...] / jnp.maximum(cnts[...], 1.0)
        out_o[...] = jnp.dot(pooled, wl[...],
                             preferred_element_type=jnp.float32) + bl[...]


def _row_spec(cols):
    return pl.BlockSpec((BLK, cols), lambda i: (i, 0))


def _full_spec(r, c):
    return pl.BlockSpec((r, c), lambda i: (0, 0))


_tc1 = pl.pallas_call(
    _tc1_body,
    grid=(TC_GRID,),
    in_specs=[_row_spec(1), _row_spec(1), _row_spec(3), _full_spec(3, F1)],
    out_specs=[_row_spec(1), _row_spec(F1), _row_spec(F1)],
    out_shape=[
        jax.ShapeDtypeStruct((NP, 1), jnp.float32),
        jax.ShapeDtypeStruct((NP, F1), jnp.float32),
        jax.ShapeDtypeStruct((NP, F1), jnp.float32),
    ],
)

_tc2 = pl.pallas_call(
    _tc2_body,
    grid=(TC_GRID,),
    in_specs=[_row_spec(F1), _row_spec(F1), _row_spec(F1), _row_spec(1),
              _full_spec(F1, F2), _full_spec(1, F2)],
    out_specs=[_row_spec(F2), _row_spec(F2)],
    out_shape=[
        jax.ShapeDtypeStruct((NP, F2), jnp.float32),
        jax.ShapeDtypeStruct((NP, F2), jnp.float32),
    ],
)

_tc3 = pl.pallas_call(
    _tc3_body,
    grid=(TC_GRID,),
    in_specs=[_row_spec(F2), _row_spec(F2), _row_spec(F2), _row_spec(1),
              _full_spec(1, F2), _row_spec(1), _full_spec(F2, FO),
              _full_spec(1, FO)],
    out_specs=_full_spec(NUM_GRAPHS, FO),
    out_shape=jax.ShapeDtypeStruct((NUM_GRAPHS, FO), jnp.float32),
    scratch_shapes=[
        pltpu.VMEM((NUM_GRAPHS, F2), jnp.float32),
        pltpu.VMEM((NUM_GRAPHS, 1), jnp.float32),
    ],
)


def kernel(x, edge_index, batch, W1, b1, W2, b2, Wl, bl):
    pad = NP - N_NODES
    xp = jnp.pad(x, ((0, pad), (0, 0)))
    batchp = jnp.pad(batch, (0, pad), constant_values=NUM_GRAPHS)
    batchp = batchp.reshape(NP, 1)
    epad = E_PAD - N_EDGES
    rows_p = jnp.concatenate(
        [edge_index[0], jnp.zeros((epad,), edge_index.dtype)])
    cols_p = jnp.concatenate(
        [edge_index[1], jnp.full((epad,), N_NODES, edge_index.dtype)])
    row3 = rows_p.reshape(NCHUNK, OPS_PER_CHUNK, 128)
    col3 = cols_p.reshape(NCHUNK, OPS_PER_CHUNK, 128)
    ones128 = jnp.ones((128,), jnp.float32)
    zeros1 = jnp.zeros((NP,), jnp.float32)
    zeros2 = jnp.zeros((NP, F1), jnp.float32)
    b1r = b1.reshape(1, F1)
    b2r = b2.reshape(1, F2)
    blr = bl.reshape(1, FO)

    cnt2 = _sc_count(col3, ones128, zeros1).reshape(NC, NP, 1)
    dinv, h1, g1 = _tc1(cnt2[0], cnt2[1], xp, W1)
    agg1 = _sc_edge(row3, col3, g1, zeros2).reshape(NC, NP, F1)
    h2, g2 = _tc2(agg1[0], agg1[1], h1, dinv, W2, b1r)
    agg2 = _sc_edge(row3, col3, g2, zeros2).reshape(NC, NP, F1)
    return _tc3(agg2[0], agg2[1], h2, dinv, b2r, batchp, Wl, blr)


# speedup vs baseline: 117.1735x; 1.4022x over previous
import jax
import jax.numpy as jnp
from jax import lax
from jax.experimental import pallas as pl
from jax.experimental.pallas import tpu as pltpu
from jax.experimental.pallas import tpu_sc as plsc

N_NODES = 100000
N_EDGES = 6400000
NUM_GRAPHS = 128
F1, F2, FO = 8, 8, 2

NC, NS = 2, 16
NW = NC * NS
NP = 100096
RPT = NP // NS

OPS_PER_CHUNK = 8
CHUNK = OPS_PER_CHUNK * 128
NCHUNK = N_EDGES // CHUNK
T_FULL = NCHUNK // NW
LAST_W = NCHUNK - T_FULL * NW
BLK = RPT
TC_GRID = NP // BLK
STAGE_N = 8
CP_R = RPT // STAGE_N


def _sc_count_body(edge_hbm, ones_hbm, zeros_hbm, out_hbm, cnt_sp, idx_v,
                   ones_v, cp_v, isem, ssem):
    cid = lax.axis_index("c")
    sid = lax.axis_index("s")
    wid = sid * NC + cid
    base = sid * RPT
    extra = wid < LAST_W
    pltpu.sync_copy(zeros_hbm.at[pl.ds(base, RPT)], cp_v)
    pltpu.sync_copy(cp_v, cnt_sp.at[pl.ds(base, RPT)])
    pltpu.sync_copy(ones_hbm, ones_v)
    plsc.subcore_barrier()

    pltpu.async_copy(edge_hbm.at[1, wid], idx_v.at[0], isem)

    def trip(k, carry):
        b = k & 1
        ck = wid + k * NW
        pltpu.make_async_copy(edge_hbm.at[1, ck], idx_v.at[b], isem).wait()

        @pl.when(k > 0)
        def _():
            for j in range(OPS_PER_CHUNK):
                pltpu.make_async_copy(
                    ones_v, cnt_sp.at[idx_v.at[1 - b, j]], ssem).wait()

        @pl.when(jnp.logical_or(k < T_FULL - 1, extra))
        def _():
            pltpu.async_copy(edge_hbm.at[1, ck + NW], idx_v.at[1 - b], isem)

        for j in range(OPS_PER_CHUNK):
            pltpu.async_copy(ones_v, cnt_sp.at[idx_v.at[b, j]], ssem,
                             add=True)
        return carry

    lax.fori_loop(0, T_FULL, trip, 0)
    bl_ = (T_FULL - 1) & 1
    for j in range(OPS_PER_CHUNK):
        pltpu.make_async_copy(ones_v, cnt_sp.at[idx_v.at[bl_, j]], ssem).wait()

    @pl.when(extra)
    def _():
        bx = T_FULL & 1
        ck = wid + T_FULL * NW
        pltpu.make_async_copy(edge_hbm.at[1, ck], idx_v.at[bx], isem).wait()
        for j in range(OPS_PER_CHUNK):
            pltpu.async_copy(ones_v, cnt_sp.at[idx_v.at[bx, j]], ssem,
                             add=True)
        for j in range(OPS_PER_CHUNK):
            pltpu.make_async_copy(ones_v, cnt_sp.at[idx_v.at[bx, j]],
                                  ssem).wait()

    plsc.subcore_barrier()
    pltpu.sync_copy(cnt_sp.at[pl.ds(base, RPT)], cp_v)
    pltpu.sync_copy(cp_v, out_hbm.at[pl.ds(cid * NP + base, RPT)])


_sc_count = pl.kernel(
    _sc_count_body,
    out_type=jax.ShapeDtypeStruct((NC * NP,), jnp.float32),
    mesh=plsc.VectorSubcoreMesh(core_axis_name="c", subcore_axis_name="s"),
    compiler_params=pltpu.CompilerParams(use_tc_tiling_on_sc=False),
    scratch_types=[
        pltpu.VMEM_SHARED((NP,), jnp.float32),
        pltpu.VMEM((2, OPS_PER_CHUNK, 128), jnp.int32),
        pltpu.VMEM((128,), jnp.float32),
        pltpu.VMEM((RPT,), jnp.float32),
        pltpu.SemaphoreType.DMA,
        pltpu.SemaphoreType.DMA,
    ],
)


def _sc_edge_body(edge_hbm, g_hbm, zeros_hbm, out_hbm, g_sp, agg_sp, row_v,
                  col_v, msg_v, cp_v, isem, gsem, ssem):
    cid = lax.axis_index("c")
    sid = lax.axis_index("s")
    wid = sid * NC + cid
    base = sid * RPT
    extra = wid < LAST_W
    for st in range(STAGE_N):
        o = base + st * CP_R
        pltpu.sync_copy(zeros_hbm.at[pl.ds(o, CP_R)], cp_v)
        pltpu.sync_copy(cp_v, agg_sp.at[pl.ds(o, CP_R)])
        pltpu.sync_copy(g_hbm.at[pl.ds(o, CP_R)], cp_v)
        pltpu.sync_copy(cp_v, g_sp.at[pl.ds(o, CP_R)])
    plsc.subcore_barrier()

    pltpu.async_copy(edge_hbm.at[0, wid], row_v.at[0], isem)
    pltpu.async_copy(edge_hbm.at[1, wid], col_v.at[0], isem)

    def trip(k, carry):
        b = k & 1
        ck = wid + k * NW
        pltpu.make_async_copy(edge_hbm.at[0, ck], row_v.at[b], isem).wait()
        pltpu.make_async_copy(edge_hbm.at[1, ck], col_v.at[b], isem).wait()

        gds = [
            pltpu.async_copy(g_sp.at[row_v.at[b, j]], msg_v.at[b, j], gsem)
            for j in range(OPS_PER_CHUNK)
        ]

        @pl.when(k > 0)
        def _():
            for j in range(OPS_PER_CHUNK):
                pltpu.make_async_copy(
                    msg_v.at[1 - b, j],
                    agg_sp.at[col_v.at[1 - b, j]], ssem).wait()

        @pl.when(jnp.logical_or(k < T_FULL - 1, extra))
        def _():
            pltpu.async_copy(edge_hbm.at[0, ck + NW], row_v.at[1 - b], isem)
            pltpu.async_copy(edge_hbm.at[1, ck + NW], col_v.at[1 - b], isem)

        for j in range(OPS_PER_CHUNK):
            gds[j].wait()
        for j in range(OPS_PER_CHUNK):
            pltpu.async_copy(msg_v.at[b, j], agg_sp.at[col_v.at[b, j]], ssem,
                             add=True)
        return carry

    lax.fori_loop(0, T_FULL, trip, 0)
    bl_ = (T_FULL - 1) & 1
    for j in range(OPS_PER_CHUNK):
        pltpu.make_async_copy(msg_v.at[bl_, j],
                              agg_sp.at[col_v.at[bl_, j]], ssem).wait()

    @pl.when(extra)
    def _():
        bx = T_FULL & 1
        ck = wid + T_FULL * NW
        pltpu.make_async_copy(edge_hbm.at[0, ck], row_v.at[bx], isem).wait()
        pltpu.make_async_copy(edge_hbm.at[1, ck], col_v.at[bx], isem).wait()
        gds = [
            pltpu.async_copy(g_sp.at[row_v.at[bx, j]], msg_v.at[bx, j], gsem)
            for j in range(OPS_PER_CHUNK)
        ]
        for j in range(OPS_PER_CHUNK):
            gds[j].wait()
        for j in range(OPS_PER_CHUNK):
            pltpu.async_copy(msg_v.at[bx, j], agg_sp.at[col_v.at[bx, j]],
                             ssem, add=True)
        for j in range(OPS_PER_CHUNK):
            pltpu.make_async_copy(msg_v.at[bx, j],
                                  agg_sp.at[col_v.at[bx, j]], ssem).wait()

    plsc.subcore_barrier()
    for st in range(STAGE_N):
        o = base + st * CP_R
        pltpu.sync_copy(agg_sp.at[pl.ds(o, CP_R)], cp_v)
        pltpu.sync_copy(cp_v, out_hbm.at[pl.ds(cid * NP + o, CP_R)])


_sc_edge = pl.kernel(
    _sc_edge_body,
    out_type=jax.ShapeDtypeStruct((NC * NP, F1), jnp.float32),
    mesh=plsc.VectorSubcoreMesh(core_axis_name="c", subcore_axis_name="s"),
    compiler_params=pltpu.CompilerParams(use_tc_tiling_on_sc=False),
    scratch_types=[
        pltpu.VMEM_SHARED((NP, F1), jnp.float32),
        pltpu.VMEM_SHARED((NP, F1), jnp.float32),
        pltpu.VMEM((2, OPS_PER_CHUNK, 128), jnp.int32),
        pltpu.VMEM((2, OPS_PER_CHUNK, 128), jnp.int32),
        pltpu.VMEM((2, OPS_PER_CHUNK, 128, F1), jnp.float32),
        pltpu.VMEM((CP_R, F1), jnp.float32),
        pltpu.SemaphoreType.DMA,
        pltpu.SemaphoreType.DMA,
        pltpu.SemaphoreType.DMA,
    ],
)


def _tc1_body(cnta, cntb, x, w1, dinv_o, h1_o, g1_o):
    cnt = cnta[...] + cntb[...] + 1.0
    dinv = lax.rsqrt(cnt)
    h1 = jnp.dot(x[...], w1[...], preferred_element_type=jnp.float32)
    dinv_o[...] = dinv
    h1_o[...] = h1
    g1_o[...] = h1 * dinv


def _tc2_body(agga, aggb, h1, dinv, w2, b1, h2_o, g2_o):
    dv = dinv[...]
    pre = (agga[...] + aggb[...]) * dv + h1[...] * (dv * dv) + b1[...]
    act = jnp.maximum(pre, 0.0)
    h2 = jnp.dot(act, w2[...], preferred_element_type=jnp.float32)
    h2_o[...] = h2
    g2_o[...] = h2 * dv


def _tc3_body(agga, aggb, h2, dinv, b2, batch, wl, bl, out_o, sums, cnts):
    i = pl.program_id(0)
    dv = dinv[...]
    z = jnp.maximum((agga[...] + aggb[...]) * dv + h2[...] * (dv * dv)
                    + b2[...], 0.0)
    onehot = (batch[...] == lax.broadcasted_iota(
        jnp.int32, (BLK, NUM_GRAPHS), 1)).astype(jnp.float32)
    dims = (((0,), (0,)), ((), ()))
    psum = lax.dot_general(onehot, z, dims,
                           preferred_element_type=jnp.float32)
    pcnt = lax.dot_general(onehot, jnp.ones((BLK, 1), jnp.float32), dims,
                           preferred_element_type=jnp.float32)

    @pl.when(i == 0)
    def _():
        sums[...] = psum
        cnts[...] = pcnt

    @pl.when(i > 0)
    def _():
        sums[...] += psum
        cnts[...] += pcnt

    @pl.when(i == TC_GRID - 1)
    def _():
        pooled = sums[...] / jnp.maximum(cnts[...], 1.0)
        out_o[...] = jnp.dot(pooled, wl[...],
                             preferred_element_type=jnp.float32) + bl[...]


def _row_spec(cols):
    return pl.BlockSpec((BLK, cols), lambda i: (i, 0))


def _full_spec(r, c):
    return pl.BlockSpec((r, c), lambda i: (0, 0))


_tc1 = pl.pallas_call(
    _tc1_body,
    grid=(TC_GRID,),
    in_specs=[_row_spec(1), _row_spec(1), _row_spec(3), _full_spec(3, F1)],
    out_specs=[_row_spec(1), _row_spec(F1), _row_spec(F1)],
    out_shape=[
        jax.ShapeDtypeStruct((NP, 1), jnp.float32),
        jax.ShapeDtypeStruct((NP, F1), jnp.float32),
        jax.ShapeDtypeStruct((NP, F1), jnp.float32),
    ],
)

_tc2 = pl.pallas_call(
    _tc2_body,
    grid=(TC_GRID,),
    in_specs=[_row_spec(F1), _row_spec(F1), _row_spec(F1), _row_spec(1),
              _full_spec(F1, F2), _full_spec(1, F2)],
    out_specs=[_row_spec(F2), _row_spec(F2)],
    out_shape=[
        jax.ShapeDtypeStruct((NP, F2), jnp.float32),
        jax.ShapeDtypeStruct((NP, F2), jnp.float32),
    ],
)

_tc3 = pl.pallas_call(
    _tc3_body,
    grid=(TC_GRID,),
    in_specs=[_row_spec(F2), _row_spec(F2), _row_spec(F2), _row_spec(1),
              _full_spec(1, F2), _row_spec(1), _full_spec(F2, FO),
              _full_spec(1, FO)],
    out_specs=_full_spec(NUM_GRAPHS, FO),
    out_shape=jax.ShapeDtypeStruct((NUM_GRAPHS, FO), jnp.float32),
    scratch_shapes=[
        pltpu.VMEM((NUM_GRAPHS, F2), jnp.float32),
        pltpu.VMEM((NUM_GRAPHS, 1), jnp.float32),
    ],
)


def kernel(x, edge_index, batch, W1, b1, W2, b2, Wl, bl):
    pad = NP - N_NODES
    xp = jnp.pad(x, ((0, pad), (0, 0)))
    batchp = jnp.pad(batch, (0, pad), constant_values=NUM_GRAPHS)
    batchp = batchp.reshape(NP, 1)
    edge4 = edge_index.reshape(2, NCHUNK, OPS_PER_CHUNK, 128)
    ones128 = jnp.ones((128,), jnp.float32)
    zeros1 = jnp.zeros((NP,), jnp.float32)
    zeros2 = jnp.zeros((NP, F1), jnp.float32)
    b1r = b1.reshape(1, F1)
    b2r = b2.reshape(1, F2)
    blr = bl.reshape(1, FO)

    cnt2 = _sc_count(edge4, ones128, zeros1).reshape(NC, NP, 1)
    dinv, h1, g1 = _tc1(cnt2[0], cnt2[1], xp, W1)
    agg1 = _sc_edge(edge4, g1, zeros2).reshape(NC, NP, F1)
    h2, g2 = _tc2(agg1[0], agg1[1], h1, dinv, W2, b1r)
    agg2 = _sc_edge(edge4, g2, zeros2).reshape(NC, NP, F1)
    return _tc3(agg2[0], agg2[1], h2, dinv, b2r, batchp, Wl, blr)


# speedup vs baseline: 141.6716x; 1.2091x over previous
import jax
import jax.numpy as jnp
from jax import lax
from jax.experimental import pallas as pl
from jax.experimental.pallas import tpu as pltpu
from jax.experimental.pallas import tpu_sc as plsc

N_NODES = 100000
N_EDGES = 6400000
NUM_GRAPHS = 128
F1, F2, FO = 8, 8, 2

NC, NS = 2, 16
NW = NC * NS
NP = 100096
RPT = NP // NS

OPS_PER_CHUNK = 8
CHUNK = OPS_PER_CHUNK * 128
NCHUNK = N_EDGES // CHUNK
T_FULL = NCHUNK // NW
LAST_W = NCHUNK - T_FULL * NW
BLK = RPT
TC_GRID = NP // BLK
STAGE_N = 8
CP_R = RPT // STAGE_N


def _sc_count_body(edge_hbm, ones_hbm, zeros_hbm, out_hbm, cnt_sp, idx_v,
                   ones_v, cp_v, isem, ssem):
    cid = lax.axis_index("c")
    sid = lax.axis_index("s")
    wid = sid * NC + cid
    base = sid * RPT
    extra = wid < LAST_W
    pltpu.sync_copy(zeros_hbm.at[pl.ds(base, RPT)], cp_v)
    pltpu.sync_copy(cp_v, cnt_sp.at[pl.ds(base, RPT)])
    pltpu.sync_copy(ones_hbm, ones_v)
    plsc.subcore_barrier()

    pltpu.async_copy(edge_hbm.at[1, wid], idx_v.at[0], isem)

    def trip(k, carry):
        b = k & 1
        ck = wid + k * NW
        pltpu.make_async_copy(edge_hbm.at[1, ck], idx_v.at[b], isem).wait()

        @pl.when(k > 0)
        def _():
            for j in range(OPS_PER_CHUNK):
                pltpu.make_async_copy(
                    ones_v, cnt_sp.at[idx_v.at[1 - b, j]], ssem).wait()

        @pl.when(jnp.logical_or(k < T_FULL - 1, extra))
        def _():
            pltpu.async_copy(edge_hbm.at[1, ck + NW], idx_v.at[1 - b], isem)

        for j in range(OPS_PER_CHUNK):
            pltpu.async_copy(ones_v, cnt_sp.at[idx_v.at[b, j]], ssem,
                             add=True)
        return carry

    lax.fori_loop(0, T_FULL, trip, 0)
    bl_ = (T_FULL - 1) & 1
    for j in range(OPS_PER_CHUNK):
        pltpu.make_async_copy(ones_v, cnt_sp.at[idx_v.at[bl_, j]], ssem).wait()

    @pl.when(extra)
    def _():
        bx = T_FULL & 1
        ck = wid + T_FULL * NW
        pltpu.make_async_copy(edge_hbm.at[1, ck], idx_v.at[bx], isem).wait()
        for j in range(OPS_PER_CHUNK):
            pltpu.async_copy(ones_v, cnt_sp.at[idx_v.at[bx, j]], ssem,
                             add=True)
        for j in range(OPS_PER_CHUNK):
            pltpu.make_async_copy(ones_v, cnt_sp.at[idx_v.at[bx, j]],
                                  ssem).wait()

    plsc.subcore_barrier()
    pltpu.sync_copy(cnt_sp.at[pl.ds(base, RPT)], cp_v)
    pltpu.sync_copy(cp_v, out_hbm.at[pl.ds(cid * NP + base, RPT)])


_sc_count = pl.kernel(
    _sc_count_body,
    out_type=jax.ShapeDtypeStruct((NC * NP,), jnp.float32),
    mesh=plsc.VectorSubcoreMesh(core_axis_name="c", subcore_axis_name="s"),
    compiler_params=pltpu.CompilerParams(use_tc_tiling_on_sc=False),
    scratch_types=[
        pltpu.VMEM_SHARED((NP,), jnp.float32),
        pltpu.VMEM((2, OPS_PER_CHUNK, 128), jnp.int32),
        pltpu.VMEM((128,), jnp.float32),
        pltpu.VMEM((RPT,), jnp.float32),
        pltpu.SemaphoreType.DMA,
        pltpu.SemaphoreType.DMA,
    ],
)


def _sc_edge_body(edge_hbm, g_hbm, zeros_hbm, out_hbm, g_sp, agg_sp, row_v,
                  col_v, msg_v, cp_v, isem, gsem, ssem):
    cid = lax.axis_index("c")
    sid = lax.axis_index("s")
    wid = sid * NC + cid
    base = sid * RPT
    extra = wid < LAST_W
    for st in range(STAGE_N):
        o = base + st * CP_R
        pltpu.sync_copy(zeros_hbm.at[pl.ds(o, CP_R)], cp_v)
        pltpu.sync_copy(cp_v, agg_sp.at[pl.ds(o, CP_R)])
        pltpu.sync_copy(g_hbm.at[pl.ds(o, CP_R)], cp_v)
        pltpu.sync_copy(cp_v, g_sp.at[pl.ds(o, CP_R)])
    plsc.subcore_barrier()

    pltpu.async_copy(edge_hbm.at[0, wid], row_v.at[0], isem)
    pltpu.async_copy(edge_hbm.at[1, wid], col_v.at[0], isem)

    def trip(k, carry):
        b = k & 1
        ck = wid + k * NW
        pltpu.make_async_copy(edge_hbm.at[0, ck], row_v.at[b], isem).wait()
        pltpu.make_async_copy(edge_hbm.at[1, ck], col_v.at[b], isem).wait()

        gds = [
            pltpu.async_copy(g_sp.at[row_v.at[b, j]], msg_v.at[b, j], gsem)
            for j in range(OPS_PER_CHUNK)
        ]

        @pl.when(k > 0)
        def _():
            for j in range(OPS_PER_CHUNK):
                pltpu.make_async_copy(
                    msg_v.at[1 - b, j],
                    agg_sp.at[col_v.at[1 - b, j]], ssem).wait()

        @pl.when(jnp.logical_or(k < T_FULL - 1, extra))
        def _():
            pltpu.async_copy(edge_hbm.at[0, ck + NW], row_v.at[1 - b], isem)
            pltpu.async_copy(edge_hbm.at[1, ck + NW], col_v.at[1 - b], isem)

        for j in range(OPS_PER_CHUNK):
            gds[j].wait()
        for j in range(OPS_PER_CHUNK):
            pltpu.async_copy(msg_v.at[b, j], agg_sp.at[col_v.at[b, j]], ssem,
                             add=True)
        return carry

    lax.fori_loop(0, T_FULL, trip, 0)
    bl_ = (T_FULL - 1) & 1
    for j in range(OPS_PER_CHUNK):
        pltpu.make_async_copy(msg_v.at[bl_, j],
                              agg_sp.at[col_v.at[bl_, j]], ssem).wait()

    @pl.when(extra)
    def _():
        bx = T_FULL & 1
        ck = wid + T_FULL * NW
        pltpu.make_async_copy(edge_hbm.at[0, ck], row_v.at[bx], isem).wait()
        pltpu.make_async_copy(edge_hbm.at[1, ck], col_v.at[bx], isem).wait()
        gds = [
            pltpu.async_copy(g_sp.at[row_v.at[bx, j]], msg_v.at[bx, j], gsem)
            for j in range(OPS_PER_CHUNK)
        ]
        for j in range(OPS_PER_CHUNK):
            gds[j].wait()
        for j in range(OPS_PER_CHUNK):
            pltpu.async_copy(msg_v.at[bx, j], agg_sp.at[col_v.at[bx, j]],
                             ssem, add=True)
        for j in range(OPS_PER_CHUNK):
            pltpu.make_async_copy(msg_v.at[bx, j],
                                  agg_sp.at[col_v.at[bx, j]], ssem).wait()

    plsc.subcore_barrier()
    for st in range(STAGE_N):
        o = base + st * CP_R
        pltpu.sync_copy(agg_sp.at[pl.ds(o, CP_R)], cp_v)
        pltpu.sync_copy(cp_v, out_hbm.at[pl.ds(cid * NP + o, CP_R)])


_sc_edge = pl.kernel(
    _sc_edge_body,
    out_type=jax.ShapeDtypeStruct((NC * NP, F1), jnp.float32),
    mesh=plsc.VectorSubcoreMesh(core_axis_name="c", subcore_axis_name="s"),
    compiler_params=pltpu.CompilerParams(use_tc_tiling_on_sc=False),
    scratch_types=[
        pltpu.VMEM_SHARED((NP, F1), jnp.float32),
        pltpu.VMEM_SHARED((NP, F1), jnp.float32),
        pltpu.VMEM((2, OPS_PER_CHUNK, 128), jnp.int32),
        pltpu.VMEM((2, OPS_PER_CHUNK, 128), jnp.int32),
        pltpu.VMEM((2, OPS_PER_CHUNK, 128, F1), jnp.float32),
        pltpu.VMEM((CP_R, F1), jnp.float32),
        pltpu.SemaphoreType.DMA,
        pltpu.SemaphoreType.DMA,
        pltpu.SemaphoreType.DMA,
    ],
)


def _tc1_body(cnta, cntb, x, w1, dinv_o, h1_o, g1_o):
    cnt = cnta[...] + cntb[...] + 1.0
    dinv = lax.rsqrt(cnt)
    h1 = jnp.dot(x[...], w1[...], preferred_element_type=jnp.float32)
    dinv_o[...] = dinv
    h1_o[...] = h1
    g1_o[...] = h1 * dinv


def _tc2_body(agga, aggb, h1, dinv, w2, b1, h2_o, g2_o):
    dv = dinv[...]
    pre = (agga[...] + aggb[...]) * dv + h1[...] * (dv * dv) + b1[...]
    act = jnp.maximum(pre, 0.0)
    h2 = jnp.dot(act, w2[...], preferred_element_type=jnp.float32)
    h2_o[...] = h2
    g2_o[...] = h2 * dv


def _tc3_body(agga, aggb, h2, dinv, b2, batch, wl, bl, out_o, sums, cnts):
    i = pl.program_id(0)
    dv = dinv[...]
    z = jnp.maximum((agga[...] + aggb[...]) * dv + h2[...] * (dv * dv)
                    + b2[...], 0.0)
    onehot = (batch[...] == lax.broadcasted_iota(
        jnp.int32, (BLK, NUM_GRAPHS), 1)).astype(jnp.float32)
    dims = (((0,), (0,)), ((), ()))
    psum = lax.dot_general(onehot, z, dims,
                           preferred_element_type=jnp.float32)
    pcnt = lax.dot_general(onehot, jnp.ones((BLK, 1), jnp.float32), dims,
                           preferred_element_type=jnp.float32)

    @pl.when(i == 0)
    def _():
        sums[...] = psum
        cnts[...] = pcnt

    @pl.when(i > 0)
    def _():
        sums[...] += psum
        cnts[...] += pcnt

    @pl.when(i == TC_GRID - 1)
    def _():
        pooled = sums[...] / jnp.maximum(cnts[...], 1.0)
        out_o[...] = jnp.dot(pooled, wl[...],
                             preferred_element_type=jnp.float32) + bl[...]


def _row_spec(cols):
    return pl.BlockSpec((BLK, cols), lambda i: (i, 0))


def _row_spec_hi(cols):
    return pl.BlockSpec((BLK, cols), lambda i: (i + TC_GRID, 0))


def _full_spec(r, c):
    return pl.BlockSpec((r, c), lambda i: (0, 0))


_tc1 = pl.pallas_call(
    _tc1_body,
    grid=(TC_GRID,),
    in_specs=[_row_spec(1), _row_spec_hi(1), _row_spec(3), _full_spec(3, F1)],
    out_specs=[_row_spec(1), _row_spec(F1), _row_spec(F1)],
    out_shape=[
        jax.ShapeDtypeStruct((NP, 1), jnp.float32),
        jax.ShapeDtypeStruct((NP, F1), jnp.float32),
        jax.ShapeDtypeStruct((NP, F1), jnp.float32),
    ],
)

_tc2 = pl.pallas_call(
    _tc2_body,
    grid=(TC_GRID,),
    in_specs=[_row_spec(F1), _row_spec_hi(F1), _row_spec(F1), _row_spec(1),
              _full_spec(F1, F2), _full_spec(1, F2)],
    out_specs=[_row_spec(F2), _row_spec(F2)],
    out_shape=[
        jax.ShapeDtypeStruct((NP, F2), jnp.float32),
        jax.ShapeDtypeStruct((NP, F2), jnp.float32),
    ],
)

_tc3 = pl.pallas_call(
    _tc3_body,
    grid=(TC_GRID,),
    in_specs=[_row_spec(F2), _row_spec_hi(F2), _row_spec(F2), _row_spec(1),
              _full_spec(1, F2), _row_spec(1), _full_spec(F2, FO),
              _full_spec(1, FO)],
    out_specs=_full_spec(NUM_GRAPHS, FO),
    out_shape=jax.ShapeDtypeStruct((NUM_GRAPHS, FO), jnp.float32),
    scratch_shapes=[
        pltpu.VMEM((NUM_GRAPHS, F2), jnp.float32),
        pltpu.VMEM((NUM_GRAPHS, 1), jnp.float32),
    ],
)


def kernel(x, edge_index, batch, W1, b1, W2, b2, Wl, bl):
    pad = NP - N_NODES
    xp = jnp.pad(x, ((0, pad), (0, 0)))
    batchp = jnp.pad(batch, (0, pad), constant_values=NUM_GRAPHS)
    batchp = batchp.reshape(NP, 1)
    edge4 = edge_index.reshape(2, NCHUNK, OPS_PER_CHUNK, 128)
    ones128 = jnp.ones((128,), jnp.float32)
    zeros1 = jnp.zeros((NP,), jnp.float32)
    zeros2 = jnp.zeros((NP, F1), jnp.float32)
    b1r = b1.reshape(1, F1)
    b2r = b2.reshape(1, F2)
    blr = bl.reshape(1, FO)

    cnt = _sc_count(edge4, ones128, zeros1).reshape(NC * NP, 1)
    dinv, h1, g1 = _tc1(cnt, cnt, xp, W1)
    agg1 = _sc_edge(edge4, g1, zeros2)
    h2, g2 = _tc2(agg1, agg1, h1, dinv, W2, b1r)
    agg2 = _sc_edge(edge4, g2, zeros2)
    return _tc3(agg2, agg2, h2, dinv, b2r, batchp, Wl, blr)


# speedup vs baseline: 227.4299x; 1.6053x over previous
import jax
import jax.numpy as jnp
from jax import lax
from jax.experimental import pallas as pl
from jax.experimental.pallas import tpu as pltpu
from jax.experimental.pallas import tpu_sc as plsc

N_NODES = 100000
N_EDGES = 6400000
NUM_GRAPHS = 128
F1, F2, FO = 8, 8, 2

NC, NS = 2, 16
NW = NC * NS
NP = 100352
RPT = NP // NS

OPS_PER_CHUNK = 8
CHUNK = OPS_PER_CHUNK * 128
NCHUNK = N_EDGES // CHUNK
T_FULL = NCHUNK // NW
LAST_W = NCHUNK - T_FULL * NW
STAGE_N = 8
CP_R = RPT // STAGE_N
PR = NP // 16
TCG = 16
PB = PR // TCG


def _sc_count_body(edge_hbm, ones_hbm, zeros_hbm, out_hbm, cnt_sp, idx_v,
                   ones_v, cp_v, isem, ssem):
    cid = lax.axis_index("c")
    sid = lax.axis_index("s")
    wid = sid * NC + cid
    base = sid * RPT
    extra = wid < LAST_W
    for st in range(STAGE_N):
        o = base + st * CP_R
        pltpu.sync_copy(zeros_hbm.at[pl.ds(o, CP_R)], cp_v)
        pltpu.sync_copy(cp_v, cnt_sp.at[pl.ds(o, CP_R)])
    pltpu.sync_copy(ones_hbm, ones_v)
    plsc.subcore_barrier()

    pltpu.async_copy(edge_hbm.at[1, wid], idx_v.at[0], isem)

    def trip(k, carry):
        b = k & 1
        ck = wid + k * NW
        pltpu.make_async_copy(edge_hbm.at[1, ck], idx_v.at[b], isem).wait()

        @pl.when(k > 0)
        def _():
            for j in range(OPS_PER_CHUNK):
                pltpu.make_async_copy(
                    ones_v, cnt_sp.at[idx_v.at[1 - b, j]], ssem).wait()

        @pl.when(jnp.logical_or(k < T_FULL - 1, extra))
        def _():
            pltpu.async_copy(edge_hbm.at[1, ck + NW], idx_v.at[1 - b], isem)

        for j in range(OPS_PER_CHUNK):
            pltpu.async_copy(ones_v, cnt_sp.at[idx_v.at[b, j]], ssem,
                             add=True)
        return carry

    lax.fori_loop(0, T_FULL, trip, 0)
    bl_ = (T_FULL - 1) & 1
    for j in range(OPS_PER_CHUNK):
        pltpu.make_async_copy(ones_v, cnt_sp.at[idx_v.at[bl_, j]], ssem).wait()

    @pl.when(extra)
    def _():
        bx = T_FULL & 1
        ck = wid + T_FULL * NW
        pltpu.make_async_copy(edge_hbm.at[1, ck], idx_v.at[bx], isem).wait()
        for j in range(OPS_PER_CHUNK):
            pltpu.async_copy(ones_v, cnt_sp.at[idx_v.at[bx, j]], ssem,
                             add=True)
        for j in range(OPS_PER_CHUNK):
            pltpu.make_async_copy(ones_v, cnt_sp.at[idx_v.at[bx, j]],
                                  ssem).wait()

    plsc.subcore_barrier()
    for st in range(STAGE_N):
        o = base + st * CP_R
        pltpu.sync_copy(cnt_sp.at[pl.ds(o, CP_R)], cp_v)
        pltpu.sync_copy(cp_v, out_hbm.at[pl.ds(cid * NP + o, CP_R)])


_sc_count = pl.kernel(
    _sc_count_body,
    out_type=jax.ShapeDtypeStruct((NC * NP, F1), jnp.float32),
    mesh=plsc.VectorSubcoreMesh(core_axis_name="c", subcore_axis_name="s"),
    compiler_params=pltpu.CompilerParams(use_tc_tiling_on_sc=False),
    scratch_types=[
        pltpu.VMEM_SHARED((NP, F1), jnp.float32),
        pltpu.VMEM((2, OPS_PER_CHUNK, 128), jnp.int32),
        pltpu.VMEM((128, F1), jnp.float32),
        pltpu.VMEM((CP_R, F1), jnp.float32),
        pltpu.SemaphoreType.DMA,
        pltpu.SemaphoreType.DMA,
    ],
)


def _sc_edge_body(edge_hbm, g_hbm, zeros_hbm, out_hbm, g_sp, agg_sp, row_v,
                  col_v, msg_v, cp_v, isem, gsem, ssem):
    cid = lax.axis_index("c")
    sid = lax.axis_index("s")
    wid = sid * NC + cid
    base = sid * RPT
    extra = wid < LAST_W
    for st in range(STAGE_N):
        o = base + st * CP_R
        pltpu.sync_copy(zeros_hbm.at[pl.ds(o, CP_R)], cp_v)
        pltpu.sync_copy(cp_v, agg_sp.at[pl.ds(o, CP_R)])
        pltpu.sync_copy(g_hbm.at[pl.ds(o, CP_R)], cp_v)
        pltpu.sync_copy(cp_v, g_sp.at[pl.ds(o, CP_R)])
    plsc.subcore_barrier()

    pltpu.async_copy(edge_hbm.at[0, wid], row_v.at[0], isem)
    pltpu.async_copy(edge_hbm.at[1, wid], col_v.at[0], isem)

    def trip(k, carry):
        b = k & 1
        ck = wid + k * NW
        pltpu.make_async_copy(edge_hbm.at[0, ck], row_v.at[b], isem).wait()
        pltpu.make_async_copy(edge_hbm.at[1, ck], col_v.at[b], isem).wait()

        gds = [
            pltpu.async_copy(g_sp.at[row_v.at[b, j]], msg_v.at[b, j], gsem)
            for j in range(OPS_PER_CHUNK)
        ]

        @pl.when(k > 0)
        def _():
            for j in range(OPS_PER_CHUNK):
                pltpu.make_async_copy(
                    msg_v.at[1 - b, j],
                    agg_sp.at[col_v.at[1 - b, j]], ssem).wait()

        @pl.when(jnp.logical_or(k < T_FULL - 1, extra))
        def _():
            pltpu.async_copy(edge_hbm.at[0, ck + NW], row_v.at[1 - b], isem)
            pltpu.async_copy(edge_hbm.at[1, ck + NW], col_v.at[1 - b], isem)

        for j in range(OPS_PER_CHUNK):
            gds[j].wait()
        for j in range(OPS_PER_CHUNK):
            pltpu.async_copy(msg_v.at[b, j], agg_sp.at[col_v.at[b, j]], ssem,
                             add=True)
        return carry

    lax.fori_loop(0, T_FULL, trip, 0)
    bl_ = (T_FULL - 1) & 1
    for j in range(OPS_PER_CHUNK):
        pltpu.make_async_copy(msg_v.at[bl_, j],
                              agg_sp.at[col_v.at[bl_, j]], ssem).wait()

    @pl.when(extra)
    def _():
        bx = T_FULL & 1
        ck = wid + T_FULL * NW
        pltpu.make_async_copy(edge_hbm.at[0, ck], row_v.at[bx], isem).wait()
        pltpu.make_async_copy(edge_hbm.at[1, ck], col_v.at[bx], isem).wait()
        gds = [
            pltpu.async_copy(g_sp.at[row_v.at[bx, j]], msg_v.at[bx, j], gsem)
            for j in range(OPS_PER_CHUNK)
        ]
        for j in range(OPS_PER_CHUNK):
            gds[j].wait()
        for j in range(OPS_PER_CHUNK):
            pltpu.async_copy(msg_v.at[bx, j], agg_sp.at[col_v.at[bx, j]],
                             ssem, add=True)
        for j in range(OPS_PER_CHUNK):
            pltpu.make_async_copy(msg_v.at[bx, j],
                                  agg_sp.at[col_v.at[bx, j]], ssem).wait()

    plsc.subcore_barrier()
    for st in range(STAGE_N):
        o = base + st * CP_R
        pltpu.sync_copy(agg_sp.at[pl.ds(o, CP_R)], cp_v)
        pltpu.sync_copy(cp_v, out_hbm.at[pl.ds(cid * NP + o, CP_R)])


_sc_edge = pl.kernel(
    _sc_edge_body,
    out_type=jax.ShapeDtypeStruct((NC * NP, F1), jnp.float32),
    mesh=plsc.VectorSubcoreMesh(core_axis_name="c", subcore_axis_name="s"),
    compiler_params=pltpu.CompilerParams(use_tc_tiling_on_sc=False),
    scratch_types=[
        pltpu.VMEM_SHARED((NP, F1), jnp.float32),
        pltpu.VMEM_SHARED((NP, F1), jnp.float32),
        pltpu.VMEM((2, OPS_PER_CHUNK, 128), jnp.int32),
        pltpu.VMEM((2, OPS_PER_CHUNK, 128), jnp.int32),
        pltpu.VMEM((2, OPS_PER_CHUNK, 128, F1), jnp.float32),
        pltpu.VMEM((CP_R, F1), jnp.float32),
        pltpu.SemaphoreType.DMA,
        pltpu.SemaphoreType.DMA,
        pltpu.SemaphoreType.DMA,
    ],
)


def _tc1_body(cnta, cntb, x, w1, dinv_o, h1_o, g1_o):
    cnt = cnta[...] + cntb[...] + 1.0
    dinv = lax.rsqrt(cnt)
    h1 = jnp.dot(x[...], w1[...], preferred_element_type=jnp.float32)
    dinv_o[...] = dinv
    h1_o[...] = h1
    g1_o[...] = h1 * dinv


def _tc2_body(agga, aggb, h1, dinv, w2, b1, h2_o, g2_o):
    dv = dinv[...]
    pre = (agga[...] + aggb[...]) * dv + h1[...] * (dv * dv) + b1[...]
    act = jnp.maximum(pre, 0.0)
    h2 = jnp.dot(act, w2[...], preferred_element_type=jnp.float32)
    h2_o[...] = h2
    g2_o[...] = h2 * dv


def _tc3_body(agga, aggb, h2, dinv, b2, batch, eye, wl, bl, out_o, sums,
              cnts):
    i = pl.program_id(0)
    dv = dinv[...]
    zpk = jnp.maximum((agga[...] + aggb[...]) * dv + h2[...] * (dv * dv)
                      + b2[...], 0.0)
    dims = (((0,), (0,)), ((), ()))
    ones_pb = jnp.ones((PB, 1), jnp.float32)
    psum = jnp.zeros((NUM_GRAPHS, F2), jnp.float32)
    pcnt = jnp.zeros((NUM_GRAPHS, 1), jnp.float32)
    for k in range(16):
        oh = (batch[:, k:k + 1] == lax.broadcasted_iota(
            jnp.int32, (PB, NUM_GRAPHS), 1)).astype(jnp.float32)
        zk = jnp.dot(zpk, eye[:, F2 * k:F2 * (k + 1)],
                     preferred_element_type=jnp.float32)
        psum += lax.dot_general(oh, zk, dims,
                                preferred_element_type=jnp.float32)
        pcnt += lax.dot_general(oh, ones_pb, dims,
                                preferred_element_type=jnp.float32)

    @pl.when(i == 0)
    def _():
        sums[...] = psum
        cnts[...] = pcnt

    @pl.when(i > 0)
    def _():
        sums[...] += psum
        cnts[...] += pcnt

    @pl.when(i == TCG - 1)
    def _():
        pooled = sums[...] / jnp.maximum(cnts[...], 1.0)
        out_o[...] = jnp.dot(pooled, wl[...],
                             preferred_element_type=jnp.float32) + bl[...]


def _pk_spec():
    return pl.BlockSpec((PB, 128), lambda i: (i, 0))


def _pk_spec_hi():
    return pl.BlockSpec((PB, 128), lambda i: (i + TCG, 0))


def _full_spec(r, c):
    return pl.BlockSpec((r, c), lambda i: (0, 0))


_PK = jax.ShapeDtypeStruct((PR, 128), jnp.float32)

_tc1 = pl.pallas_call(
    _tc1_body,
    grid=(TCG,),
    in_specs=[_pk_spec(), _pk_spec_hi(), _pk_spec(), _full_spec(128, 128)],
    out_specs=[_pk_spec(), _pk_spec(), _pk_spec()],
    out_shape=[_PK, _PK, _PK],
)

_tc2 = pl.pallas_call(
    _tc2_body,
    grid=(TCG,),
    in_specs=[_pk_spec(), _pk_spec_hi(), _pk_spec(), _pk_spec(),
              _full_spec(128, 128), _full_spec(1, 128)],
    out_specs=[_pk_spec(), _pk_spec()],
    out_shape=[_PK, _PK],
)

_tc3 = pl.pallas_call(
    _tc3_body,
    grid=(TCG,),
    in_specs=[_pk_spec(), _pk_spec_hi(), _pk_spec(), _pk_spec(),
              _full_spec(1, 128), pl.BlockSpec((PB, 16), lambda i: (i, 0)),
              _full_spec(128, 128), _full_spec(F2, FO), _full_spec(1, FO)],
    out_specs=_full_spec(NUM_GRAPHS, FO),
    out_shape=jax.ShapeDtypeStruct((NUM_GRAPHS, FO), jnp.float32),
    scratch_shapes=[
        pltpu.VMEM((NUM_GRAPHS, F2), jnp.float32),
        pltpu.VMEM((NUM_GRAPHS, 1), jnp.float32),
    ],
)


def kernel(x, edge_index, batch, W1, b1, W2, b2, Wl, bl):
    pad = NP - N_NODES
    xp8 = jnp.pad(x, ((0, pad), (0, F1 - 3)))
    xpk = xp8.reshape(PR, 128)
    batchp = jnp.pad(batch, (0, pad), constant_values=NUM_GRAPHS)
    batch16 = batchp.reshape(PR, 16)
    edge4 = edge_index.reshape(2, NCHUNK, OPS_PER_CHUNK, 128)
    ones8 = jnp.ones((128, F1), jnp.float32)
    zeros2 = jnp.zeros((NP, F1), jnp.float32)
    eye16 = jnp.eye(16, dtype=jnp.float32)
    eye128 = jnp.eye(128, dtype=jnp.float32)
    w1bd = jnp.kron(eye16, jnp.pad(W1, ((0, F1 - 3), (0, 0))))
    w2bd = jnp.kron(eye16, W2)
    b1pk = jnp.tile(b1, 16).reshape(1, 128)
    b2pk = jnp.tile(b2, 16).reshape(1, 128)
    blr = bl.reshape(1, FO)

    cnt8 = _sc_count(edge4, ones8, zeros2).reshape(NC * PR, 128)
    dinvpk, h1pk, g1pk = _tc1(cnt8, cnt8, xpk, w1bd)
    agg1 = _sc_edge(edge4, g1pk.reshape(NP, F1), zeros2).reshape(NC * PR, 128)
    h2pk, g2pk = _tc2(agg1, agg1, h1pk, dinvpk, w2bd, b1pk)
    agg2 = _sc_edge(edge4, g2pk.reshape(NP, F1), zeros2).reshape(NC * PR, 128)
    return _tc3(agg2, agg2, h2pk, dinvpk, b2pk, batch16, eye128, Wl, blr)
